# baseline (speedup 1.0000x reference)
import functools

import jax
import jax.numpy as jnp
from jax import lax
from jax.experimental import pallas as pl
from jax.experimental.pallas import tpu as pltpu
from jax.experimental.pallas import tpu_sc as plsc

NC = 2
NS = 16
NW = NC * NS
EB = 128


def _sc_mesh():
    return plsc.VectorSubcoreMesh(core_axis_name="c", subcore_axis_name="s")


def _make_deg_kernel(npad, kb, d):
    rpt = npad // NS

    @functools.partial(
        pl.kernel,
        out_type=jax.ShapeDtypeStruct((NC, npad, d), jnp.float32),
        mesh=_sc_mesh(),
        scratch_types=[
            pltpu.VMEM((kb, EB), jnp.int32),
            pltpu.VMEM((EB, d), jnp.float32),
            pltpu.VMEM_SHARED((npad, d), jnp.float32),
        ],
    )
    def deg_kernel(col_hbm, out_hbm, col_v, buf_v, acc_sh):
        c = lax.axis_index("c")
        s = lax.axis_index("s")
        wid = c * NS + s

        @pl.loop(0, EB)
        def _(i):
            for j in range(d // 16):
                buf_v[i, pl.ds(j * 16, 16)] = jnp.zeros((16,), jnp.float32)

        base = s * rpt
        nfull = rpt // EB
        rem = rpt - nfull * EB

        @pl.loop(0, nfull)
        def _(k):
            pltpu.sync_copy(buf_v, acc_sh.at[pl.ds(base + k * EB, EB)])

        if rem:
            pltpu.sync_copy(buf_v.at[pl.ds(0, rem)],
                            acc_sh.at[pl.ds(base + nfull * EB, rem)])

        @pl.loop(0, EB)
        def _(i):
            for j in range(d // 16):
                buf_v[i, pl.ds(j * 16, 16)] = jnp.ones((16,), jnp.float32)

        pltpu.sync_copy(col_hbm.at[wid], col_v)
        plsc.subcore_barrier()

        @pl.loop(0, kb)
        def _(j):
            pltpu.sync_copy(buf_v, acc_sh.at[col_v.at[j]], add=True)

        plsc.subcore_barrier()

        @pl.loop(0, nfull)
        def _(k):
            pltpu.sync_copy(acc_sh.at[pl.ds(base + k * EB, EB)],
                            out_hbm.at[c, pl.ds(base + k * EB, EB)])

        if rem:
            pltpu.sync_copy(acc_sh.at[pl.ds(base + nfull * EB, rem)],
                            out_hbm.at[c, pl.ds(base + nfull * EB, rem)])

    return deg_kernel


IB = 16


def _make_layer_kernel(npad, kb, d):
    rpt = npad // NS
    nch = kb // IB

    @functools.partial(
        pl.kernel,
        out_type=jax.ShapeDtypeStruct((NC, npad, d), jnp.float32),
        mesh=_sc_mesh(),
        scratch_types=[
            pltpu.VMEM((IB, EB), jnp.int32),
            pltpu.VMEM((IB, EB), jnp.int32),
            pltpu.VMEM((EB, d), jnp.float32),
            pltpu.VMEM((EB, d), jnp.float32),
            pltpu.SemaphoreType.DMA,
            pltpu.SemaphoreType.DMA,
            pltpu.VMEM_SHARED((npad, d), jnp.float32),
        ],
    )
    def layer_kernel(g_hbm, row_hbm, col_hbm, out_hbm,
                     row_v, col_v, buf0, buf1, sem0, sem1, acc_sh):
        c = lax.axis_index("c")
        s = lax.axis_index("s")
        wid = c * NS + s

        @pl.loop(0, EB)
        def _(i):
            for j in range(d // 16):
                buf0[i, pl.ds(j * 16, 16)] = jnp.zeros((16,), jnp.float32)

        base = s * rpt
        nfull = rpt // EB
        rem = rpt - nfull * EB

        @pl.loop(0, nfull)
        def _(k):
            pltpu.sync_copy(buf0, acc_sh.at[pl.ds(base + k * EB, EB)])

        if rem:
            pltpu.sync_copy(buf0.at[pl.ds(0, rem)],
                            acc_sh.at[pl.ds(base + nfull * EB, rem)])

        plsc.subcore_barrier()

        @pl.loop(0, nch)
        def _(ci):
            pltpu.sync_copy(row_hbm.at[wid, pl.ds(ci * IB, IB)], row_v)
            pltpu.sync_copy(col_hbm.at[wid, pl.ds(ci * IB, IB)], col_v)

            pltpu.async_copy(g_hbm.at[row_v.at[0]], buf0, sem0)
            pltpu.async_copy(g_hbm.at[row_v.at[1]], buf1, sem1)

            @pl.loop(0, IB, step=2)
            def _(j):
                pltpu.make_async_copy(g_hbm.at[row_v.at[j]], buf0, sem0).wait()
                pltpu.sync_copy(buf0, acc_sh.at[col_v.at[j]], add=True)

                @pl.when(j + 2 < IB)
                def _():
                    pltpu.async_copy(g_hbm.at[row_v.at[j + 2]], buf0, sem0)

                pltpu.make_async_copy(g_hbm.at[row_v.at[j + 1]], buf1, sem1).wait()
                pltpu.sync_copy(buf1, acc_sh.at[col_v.at[j + 1]], add=True)

                @pl.when(j + 3 < IB)
                def _():
                    pltpu.async_copy(g_hbm.at[row_v.at[j + 3]], buf1, sem1)

        plsc.subcore_barrier()

        @pl.loop(0, nfull)
        def _(k):
            pltpu.sync_copy(acc_sh.at[pl.ds(base + k * EB, EB)],
                            out_hbm.at[c, pl.ds(base + k * EB, EB)])

        if rem:
            pltpu.sync_copy(acc_sh.at[pl.ds(base + nfull * EB, rem)],
                            out_hbm.at[c, pl.ds(base + nfull * EB, rem)])

    return layer_kernel


def _deg_combine_body(pa_ref, pb_ref, o_ref):
    o_ref[...] = lax.rsqrt(pa_ref[...] + pb_ref[...] + 1.0)


def _tc_first_body(x_ref, dv_ref, w_ref, o_ref):
    u = jnp.dot(x_ref[...], w_ref[...], preferred_element_type=jnp.float32)
    o_ref[...] = u * dv_ref[...]


def _tc_mid_body(sa_ref, sb_ref, g_ref, dv_ref, b_ref, w_ref, o_ref):
    dinv = dv_ref[...]
    h = dinv * (sa_ref[...] + sb_ref[...] + g_ref[...]) + b_ref[...]
    h = jnp.maximum(h, 0.0)
    o_ref[...] = dinv * jnp.dot(h, w_ref[...], preferred_element_type=jnp.float32)


def _tc_last_body(sa_ref, sb_ref, g_ref, dv_ref, b_ref, o_ref):
    o_ref[...] = dv_ref[...] * (sa_ref[...] + sb_ref[...] + g_ref[...]) + b_ref[...]


def _row_spec(r, d):
    return pl.BlockSpec((r, d), lambda i: (i, 0))


def _full_spec(shape):
    return pl.BlockSpec(shape, lambda i: tuple(0 for _ in shape))


def _tc_call(body, in_specs, grid, npad, d, args):
    return pl.pallas_call(
        body,
        grid=(grid,),
        in_specs=in_specs,
        out_specs=_row_spec(npad // grid, d),
        out_shape=jax.ShapeDtypeStruct((npad, d), jnp.float32),
    )(*args)


def kernel(x, edge_index, W1, b1, W2, b2, W3, b3):
    n, d_in = x.shape
    e = edge_index.shape[1]
    d = W1.shape[1]

    npad = ((n + 1 + 127) // 128) * 128
    e_pad = ((e + NW * EB * IB - 1) // (NW * EB * IB)) * (NW * EB * IB)
    kb = e_pad // (NW * EB)

    row = edge_index[0]
    col = edge_index[1]
    pad = e_pad - e
    row_p = jnp.concatenate([row, jnp.zeros((pad,), jnp.int32)]).reshape(NW, kb, EB)
    col_p = jnp.concatenate([col, jnp.full((pad,), n, jnp.int32)]).reshape(NW, kb, EB)
    x_p = jnp.pad(x, ((0, npad - n), (0, 0)))

    deg_kernel = _make_deg_kernel(npad, kb, d)
    layer_kernel = _make_layer_kernel(npad, kb, d)

    degp = deg_kernel(col_p)

    dinv_b = pl.pallas_call(
        _deg_combine_body,
        grid=(8,),
        in_specs=[_row_spec(npad // 8, d), _row_spec(npad // 8, d)],
        out_specs=_row_spec(npad // 8, d),
        out_shape=jax.ShapeDtypeStruct((npad, d), jnp.float32),
    )(degp[0], degp[1])

    grid = 8
    r = npad // grid
    b1r = b1.reshape(1, d)
    b2r = b2.reshape(1, d)
    b3r = b3.reshape(1, d)

    dv_spec = _row_spec(r, 128)
    w_spec = _full_spec((d_in, d))
    b_spec = _full_spec((1, d))

    g1 = _tc_call(_tc_first_body,
                  [_row_spec(r, d_in), dv_spec, w_spec],
                  grid, npad, d, (x_p, dinv_b, W1))
    s1 = layer_kernel(g1, row_p, col_p)
    g2 = _tc_call(_tc_mid_body,
                  [_row_spec(r, d), _row_spec(r, d), _row_spec(r, d),
                   dv_spec, b_spec, w_spec],
                  grid, npad, d, (s1[0], s1[1], g1, dinv_b, b1r, W2))
    s2 = layer_kernel(g2, row_p, col_p)
    g3 = _tc_call(_tc_mid_body,
                  [_row_spec(r, d), _row_spec(r, d), _row_spec(r, d),
                   dv_spec, b_spec, w_spec],
                  grid, npad, d, (s2[0], s2[1], g2, dinv_b, b2r, W3))
    s3 = layer_kernel(g3, row_p, col_p)
    out = _tc_call(_tc_last_body,
                   [_row_spec(r, d), _row_spec(r, d), _row_spec(r, d),
                    dv_spec, b_spec],
                   grid, npad, d, (s3[0], s3[1], g3, dinv_b, b3r))
    return out[:n]


# speedup vs baseline: 1.0132x; 1.0132x over previous
import functools

import jax
import jax.numpy as jnp
from jax import lax
from jax.experimental import pallas as pl
from jax.experimental.pallas import tpu as pltpu
from jax.experimental.pallas import tpu_sc as plsc

NC = 2
NS = 16
NW = NC * NS
EB = 128


def _sc_mesh():
    return plsc.VectorSubcoreMesh(core_axis_name="c", subcore_axis_name="s")


def _make_deg_kernel(npad, kb, d):
    rpt = npad // NS

    @functools.partial(
        pl.kernel,
        out_type=jax.ShapeDtypeStruct((NC, npad, d), jnp.float32),
        mesh=_sc_mesh(),
        scratch_types=[
            pltpu.VMEM((kb, EB), jnp.int32),
            pltpu.VMEM((EB, d), jnp.float32),
            pltpu.VMEM_SHARED((npad, d), jnp.float32),
        ],
    )
    def deg_kernel(col_hbm, out_hbm, col_v, buf_v, acc_sh):
        c = lax.axis_index("c")
        s = lax.axis_index("s")
        wid = c * NS + s

        @pl.loop(0, EB)
        def _(i):
            for j in range(d // 16):
                buf_v[i, pl.ds(j * 16, 16)] = jnp.zeros((16,), jnp.float32)

        base = s * rpt
        nfull = rpt // EB
        rem = rpt - nfull * EB

        @pl.loop(0, nfull)
        def _(k):
            pltpu.sync_copy(buf_v, acc_sh.at[pl.ds(base + k * EB, EB)])

        if rem:
            pltpu.sync_copy(buf_v.at[pl.ds(0, rem)],
                            acc_sh.at[pl.ds(base + nfull * EB, rem)])

        @pl.loop(0, EB)
        def _(i):
            for j in range(d // 16):
                buf_v[i, pl.ds(j * 16, 16)] = jnp.ones((16,), jnp.float32)

        pltpu.sync_copy(col_hbm.at[wid], col_v)
        plsc.subcore_barrier()

        @pl.loop(0, kb)
        def _(j):
            pltpu.sync_copy(buf_v, acc_sh.at[col_v.at[j]], add=True)

        plsc.subcore_barrier()

        @pl.loop(0, nfull)
        def _(k):
            pltpu.sync_copy(acc_sh.at[pl.ds(base + k * EB, EB)],
                            out_hbm.at[c, pl.ds(base + k * EB, EB)])

        if rem:
            pltpu.sync_copy(acc_sh.at[pl.ds(base + nfull * EB, rem)],
                            out_hbm.at[c, pl.ds(base + nfull * EB, rem)])

    return deg_kernel


IB = 40


def _make_layer_kernel(npad, kb, d):
    rpt = npad // NS
    nch = kb // IB

    @functools.partial(
        pl.kernel,
        out_type=jax.ShapeDtypeStruct((NC, npad, d), jnp.float32),
        mesh=_sc_mesh(),
        scratch_types=[
            pltpu.VMEM((IB, EB), jnp.int32),
            pltpu.VMEM((IB, EB), jnp.int32),
            pltpu.VMEM((EB, d), jnp.float32),
            pltpu.VMEM((EB, d), jnp.float32),
            pltpu.SemaphoreType.DMA,
            pltpu.SemaphoreType.DMA,
            pltpu.VMEM_SHARED((npad, d), jnp.float32),
        ],
    )
    def layer_kernel(g_hbm, row_hbm, col_hbm, out_hbm,
                     row_v, col_v, buf0, buf1, sem0, sem1, acc_sh):
        c = lax.axis_index("c")
        s = lax.axis_index("s")
        wid = c * NS + s

        @pl.loop(0, EB)
        def _(i):
            for j in range(d // 16):
                buf0[i, pl.ds(j * 16, 16)] = jnp.zeros((16,), jnp.float32)

        base = s * rpt
        nfull = rpt // EB
        rem = rpt - nfull * EB

        @pl.loop(0, nfull)
        def _(k):
            pltpu.sync_copy(buf0, acc_sh.at[pl.ds(base + k * EB, EB)])

        if rem:
            pltpu.sync_copy(buf0.at[pl.ds(0, rem)],
                            acc_sh.at[pl.ds(base + nfull * EB, rem)])

        plsc.subcore_barrier()

        @pl.loop(0, nch)
        def _(ci):
            pltpu.sync_copy(row_hbm.at[wid, pl.ds(ci * IB, IB)], row_v)
            pltpu.sync_copy(col_hbm.at[wid, pl.ds(ci * IB, IB)], col_v)

            pltpu.async_copy(g_hbm.at[row_v.at[0]], buf0, sem0)
            pltpu.async_copy(g_hbm.at[row_v.at[1]], buf1, sem1)

            @pl.loop(0, IB - 2, step=2)
            def _(j):
                pltpu.make_async_copy(g_hbm.at[row_v.at[j]], buf0, sem0).wait()
                pltpu.sync_copy(buf0, acc_sh.at[col_v.at[j]], add=True)
                pltpu.async_copy(g_hbm.at[row_v.at[j + 2]], buf0, sem0)
                pltpu.make_async_copy(g_hbm.at[row_v.at[j + 1]], buf1, sem1).wait()
                pltpu.sync_copy(buf1, acc_sh.at[col_v.at[j + 1]], add=True)
                pltpu.async_copy(g_hbm.at[row_v.at[j + 3]], buf1, sem1)

            pltpu.make_async_copy(g_hbm.at[row_v.at[IB - 2]], buf0, sem0).wait()
            pltpu.sync_copy(buf0, acc_sh.at[col_v.at[IB - 2]], add=True)
            pltpu.make_async_copy(g_hbm.at[row_v.at[IB - 1]], buf1, sem1).wait()
            pltpu.sync_copy(buf1, acc_sh.at[col_v.at[IB - 1]], add=True)

        plsc.subcore_barrier()

        @pl.loop(0, nfull)
        def _(k):
            pltpu.sync_copy(acc_sh.at[pl.ds(base + k * EB, EB)],
                            out_hbm.at[c, pl.ds(base + k * EB, EB)])

        if rem:
            pltpu.sync_copy(acc_sh.at[pl.ds(base + nfull * EB, rem)],
                            out_hbm.at[c, pl.ds(base + nfull * EB, rem)])

    return layer_kernel


def _deg_combine_body(pa_ref, pb_ref, o_ref):
    o_ref[...] = lax.rsqrt(pa_ref[...] + pb_ref[...] + 1.0)


def _tc_first_body(x_ref, dv_ref, w_ref, o_ref):
    u = jnp.dot(x_ref[...], w_ref[...], preferred_element_type=jnp.float32)
    o_ref[...] = u * dv_ref[...]


def _tc_mid_body(sa_ref, sb_ref, g_ref, dv_ref, b_ref, w_ref, o_ref):
    dinv = dv_ref[...]
    h = dinv * (sa_ref[...] + sb_ref[...] + g_ref[...]) + b_ref[...]
    h = jnp.maximum(h, 0.0)
    o_ref[...] = dinv * jnp.dot(h, w_ref[...], preferred_element_type=jnp.float32)


def _tc_last_body(sa_ref, sb_ref, g_ref, dv_ref, b_ref, o_ref):
    o_ref[...] = dv_ref[...] * (sa_ref[...] + sb_ref[...] + g_ref[...]) + b_ref[...]


def _row_spec(r, d):
    return pl.BlockSpec((r, d), lambda i: (i, 0))


def _full_spec(shape):
    return pl.BlockSpec(shape, lambda i: tuple(0 for _ in shape))


def _tc_call(body, in_specs, grid, npad, d, args):
    return pl.pallas_call(
        body,
        grid=(grid,),
        in_specs=in_specs,
        out_specs=_row_spec(npad // grid, d),
        out_shape=jax.ShapeDtypeStruct((npad, d), jnp.float32),
    )(*args)


def kernel(x, edge_index, W1, b1, W2, b2, W3, b3):
    n, d_in = x.shape
    e = edge_index.shape[1]
    d = W1.shape[1]

    npad = ((n + 1 + 127) // 128) * 128
    e_pad = ((e + NW * EB * IB - 1) // (NW * EB * IB)) * (NW * EB * IB)
    kb = e_pad // (NW * EB)

    row = edge_index[0]
    col = edge_index[1]
    pad = e_pad - e
    row_p = jnp.concatenate([row, jnp.zeros((pad,), jnp.int32)]).reshape(NW, kb, EB)
    col_p = jnp.concatenate([col, jnp.full((pad,), n, jnp.int32)]).reshape(NW, kb, EB)
    x_p = jnp.pad(x, ((0, npad - n), (0, 0)))

    deg_kernel = _make_deg_kernel(npad, kb, d)
    layer_kernel = _make_layer_kernel(npad, kb, d)

    degp = deg_kernel(col_p)

    dinv_b = pl.pallas_call(
        _deg_combine_body,
        grid=(8,),
        in_specs=[_row_spec(npad // 8, d), _row_spec(npad // 8, d)],
        out_specs=_row_spec(npad // 8, d),
        out_shape=jax.ShapeDtypeStruct((npad, d), jnp.float32),
    )(degp[0], degp[1])

    grid = 8
    r = npad // grid
    b1r = b1.reshape(1, d)
    b2r = b2.reshape(1, d)
    b3r = b3.reshape(1, d)

    dv_spec = _row_spec(r, 128)
    w_spec = _full_spec((d_in, d))
    b_spec = _full_spec((1, d))

    g1 = _tc_call(_tc_first_body,
                  [_row_spec(r, d_in), dv_spec, w_spec],
                  grid, npad, d, (x_p, dinv_b, W1))
    s1 = layer_kernel(g1, row_p, col_p)
    g2 = _tc_call(_tc_mid_body,
                  [_row_spec(r, d), _row_spec(r, d), _row_spec(r, d),
                   dv_spec, b_spec, w_spec],
                  grid, npad, d, (s1[0], s1[1], g1, dinv_b, b1r, W2))
    s2 = layer_kernel(g2, row_p, col_p)
    g3 = _tc_call(_tc_mid_body,
                  [_row_spec(r, d), _row_spec(r, d), _row_spec(r, d),
                   dv_spec, b_spec, w_spec],
                  grid, npad, d, (s2[0], s2[1], g2, dinv_b, b2r, W3))
    s3 = layer_kernel(g3, row_p, col_p)
    out = _tc_call(_tc_last_body,
                   [_row_spec(r, d), _row_spec(r, d), _row_spec(r, d),
                    dv_spec, b_spec],
                   grid, npad, d, (s3[0], s3[1], g3, dinv_b, b3r))
    return out[:n]


# speedup vs baseline: 1.4618x; 1.4429x over previous
import functools

import jax
import jax.numpy as jnp
from jax import lax
from jax.experimental import pallas as pl
from jax.experimental.pallas import tpu as pltpu
from jax.experimental.pallas import tpu_sc as plsc

NC = 2
NS = 16
NW = NC * NS
EB = 128


def _sc_mesh():
    return plsc.VectorSubcoreMesh(core_axis_name="c", subcore_axis_name="s")


def _make_deg_kernel(npad, kb, d):
    rpt = npad // NS

    @functools.partial(
        pl.kernel,
        out_type=jax.ShapeDtypeStruct((NC, npad, d), jnp.float32),
        mesh=_sc_mesh(),
        scratch_types=[
            pltpu.VMEM((kb, EB), jnp.int32),
            pltpu.VMEM((EB, d), jnp.float32),
            pltpu.VMEM_SHARED((npad, d), jnp.float32),
        ],
    )
    def deg_kernel(col_hbm, out_hbm, col_v, buf_v, acc_sh):
        c = lax.axis_index("c")
        s = lax.axis_index("s")
        wid = c * NS + s

        @pl.loop(0, EB)
        def _(i):
            for j in range(d // 16):
                buf_v[i, pl.ds(j * 16, 16)] = jnp.zeros((16,), jnp.float32)

        base = s * rpt
        nfull = rpt // EB
        rem = rpt - nfull * EB

        @pl.loop(0, nfull)
        def _(k):
            pltpu.sync_copy(buf_v, acc_sh.at[pl.ds(base + k * EB, EB)])

        if rem:
            pltpu.sync_copy(buf_v.at[pl.ds(0, rem)],
                            acc_sh.at[pl.ds(base + nfull * EB, rem)])

        @pl.loop(0, EB)
        def _(i):
            for j in range(d // 16):
                buf_v[i, pl.ds(j * 16, 16)] = jnp.ones((16,), jnp.float32)

        pltpu.sync_copy(col_hbm.at[wid], col_v)
        plsc.subcore_barrier()

        @pl.loop(0, kb)
        def _(j):
            pltpu.sync_copy(buf_v, acc_sh.at[col_v.at[j]], add=True)

        plsc.subcore_barrier()

        @pl.loop(0, nfull)
        def _(k):
            pltpu.sync_copy(acc_sh.at[pl.ds(base + k * EB, EB)],
                            out_hbm.at[c, pl.ds(base + k * EB, EB)])

        if rem:
            pltpu.sync_copy(acc_sh.at[pl.ds(base + nfull * EB, rem)],
                            out_hbm.at[c, pl.ds(base + nfull * EB, rem)])

    return deg_kernel


def _make_layer_kernel(npad, kb, d):
    rpt = npad // NS

    @functools.partial(
        pl.kernel,
        out_type=jax.ShapeDtypeStruct((NC, npad, d), jnp.float32),
        mesh=_sc_mesh(),
        scratch_types=[
            pltpu.VMEM((kb, EB), jnp.int32),
            pltpu.VMEM((kb, EB), jnp.int32),
            pltpu.VMEM((EB, d), jnp.float32),
            pltpu.VMEM_SHARED((npad, d), jnp.float32),
        ],
    )
    def layer_kernel(g_hbm, row_hbm, col_hbm, out_hbm, row_v, col_v, buf_v, acc_sh):
        c = lax.axis_index("c")
        s = lax.axis_index("s")
        wid = c * NS + s

        @pl.loop(0, EB)
        def _(i):
            for j in range(d // 16):
                buf_v[i, pl.ds(j * 16, 16)] = jnp.zeros((16,), jnp.float32)

        base = s * rpt
        nfull = rpt // EB
        rem = rpt - nfull * EB

        @pl.loop(0, nfull)
        def _(k):
            pltpu.sync_copy(buf_v, acc_sh.at[pl.ds(base + k * EB, EB)])

        if rem:
            pltpu.sync_copy(buf_v.at[pl.ds(0, rem)],
                            acc_sh.at[pl.ds(base + nfull * EB, rem)])

        pltpu.sync_copy(row_hbm.at[wid], row_v)
        pltpu.sync_copy(col_hbm.at[wid], col_v)
        plsc.subcore_barrier()

        @pl.loop(0, kb)
        def _(j):
            pltpu.sync_copy(g_hbm.at[row_v.at[j]], buf_v)
            pltpu.sync_copy(buf_v, acc_sh.at[col_v.at[j]], add=True)

        plsc.subcore_barrier()

        @pl.loop(0, nfull)
        def _(k):
            pltpu.sync_copy(acc_sh.at[pl.ds(base + k * EB, EB)],
                            out_hbm.at[c, pl.ds(base + k * EB, EB)])

        if rem:
            pltpu.sync_copy(acc_sh.at[pl.ds(base + nfull * EB, rem)],
                            out_hbm.at[c, pl.ds(base + nfull * EB, rem)])

    return layer_kernel


def _deg_combine_body(pa_ref, pb_ref, u_ref, dv_ref, g_ref):
    dinv = lax.rsqrt(pa_ref[...] + pb_ref[...] + 1.0)
    dv_ref[...] = dinv
    g_ref[...] = dinv * u_ref[...]


def _tc_mm_body(x_ref, w_ref, o_ref):
    o_ref[...] = jnp.dot(x_ref[...], w_ref[...],
                         preferred_element_type=jnp.float32)


def _tc_first_body(x_ref, dv_ref, w_ref, o_ref):
    u = jnp.dot(x_ref[...], w_ref[...], preferred_element_type=jnp.float32)
    o_ref[...] = u * dv_ref[...]


def _tc_mid_body(sa_ref, sb_ref, g_ref, dv_ref, b_ref, w_ref, o_ref):
    dinv = dv_ref[...]
    h = dinv * (sa_ref[...] + sb_ref[...] + g_ref[...]) + b_ref[...]
    h = jnp.maximum(h, 0.0)
    o_ref[...] = dinv * jnp.dot(h, w_ref[...], preferred_element_type=jnp.float32)


def _tc_last_body(sa_ref, sb_ref, g_ref, dv_ref, b_ref, o_ref):
    o_ref[...] = dv_ref[...] * (sa_ref[...] + sb_ref[...] + g_ref[...]) + b_ref[...]


def _row_spec(r, d):
    return pl.BlockSpec((r, d), lambda i: (i, 0))


def _full_spec(shape):
    return pl.BlockSpec(shape, lambda i: tuple(0 for _ in shape))


def _tc_call(body, in_specs, grid, npad, d, args):
    return pl.pallas_call(
        body,
        grid=(grid,),
        in_specs=in_specs,
        out_specs=_row_spec(npad // grid, d),
        out_shape=jax.ShapeDtypeStruct((npad, d), jnp.float32),
    )(*args)


def kernel(x, edge_index, W1, b1, W2, b2, W3, b3):
    n, d_in = x.shape
    e = edge_index.shape[1]
    d = W1.shape[1]

    npad = ((n + 1 + 127) // 128) * 128
    e_pad = ((e + NW * EB - 1) // (NW * EB)) * (NW * EB)
    kb = e_pad // (NW * EB)

    row = edge_index[0]
    col = edge_index[1]
    pad = e_pad - e
    row_p = jnp.concatenate([row, jnp.zeros((pad,), jnp.int32)]).reshape(NW, kb, EB)
    col_p = jnp.concatenate([col, jnp.full((pad,), n, jnp.int32)]).reshape(NW, kb, EB)
    x_p = jnp.pad(x, ((0, npad - n), (0, 0)))

    deg_kernel = _make_deg_kernel(npad, kb, d)
    layer_kernel = _make_layer_kernel(npad, kb, d)

    u1 = pl.pallas_call(
        _tc_mm_body,
        grid=(8,),
        in_specs=[_row_spec(npad // 8, d_in), _full_spec((d_in, d))],
        out_specs=_row_spec(npad // 8, d),
        out_shape=jax.ShapeDtypeStruct((npad, d), jnp.float32),
    )(x_p, W1)

    degp = deg_kernel(col_p)

    dinv_b, g1 = pl.pallas_call(
        _deg_combine_body,
        grid=(8,),
        in_specs=[_row_spec(npad // 8, d), _row_spec(npad // 8, d),
                  _row_spec(npad // 8, d)],
        out_specs=[_row_spec(npad // 8, d), _row_spec(npad // 8, d)],
        out_shape=[jax.ShapeDtypeStruct((npad, d), jnp.float32),
                   jax.ShapeDtypeStruct((npad, d), jnp.float32)],
    )(degp[0], degp[1], u1)

    grid = 8
    r = npad // grid
    b1r = b1.reshape(1, d)
    b2r = b2.reshape(1, d)
    b3r = b3.reshape(1, d)

    dv_spec = _row_spec(r, 128)
    w_spec = _full_spec((d_in, d))
    b_spec = _full_spec((1, d))

    s1 = layer_kernel(g1, row_p, col_p)
    g2 = _tc_call(_tc_mid_body,
                  [_row_spec(r, d), _row_spec(r, d), _row_spec(r, d),
                   dv_spec, b_spec, w_spec],
                  grid, npad, d, (s1[0], s1[1], g1, dinv_b, b1r, W2))
    s2 = layer_kernel(g2, row_p, col_p)
    g3 = _tc_call(_tc_mid_body,
                  [_row_spec(r, d), _row_spec(r, d), _row_spec(r, d),
                   dv_spec, b_spec, w_spec],
                  grid, npad, d, (s2[0], s2[1], g2, dinv_b, b2r, W3))
    s3 = layer_kernel(g3, row_p, col_p)
    out = _tc_call(_tc_last_body,
                   [_row_spec(r, d), _row_spec(r, d), _row_spec(r, d),
                    dv_spec, b_spec],
                   grid, npad, d, (s3[0], s3[1], g3, dinv_b, b3r))
    return out[:n]


# speedup vs baseline: 1.4626x; 1.0005x over previous
import functools

import jax
import jax.numpy as jnp
from jax import lax
from jax.experimental import pallas as pl
from jax.experimental.pallas import tpu as pltpu
from jax.experimental.pallas import tpu_sc as plsc

NC = 2
NS = 16
NW = NC * NS
EB = 128


def _sc_mesh():
    return plsc.VectorSubcoreMesh(core_axis_name="c", subcore_axis_name="s")


def _make_deg_kernel(npad, kb, d):
    rpt = npad // NS

    @functools.partial(
        pl.kernel,
        out_type=jax.ShapeDtypeStruct((NC, npad, d), jnp.float32),
        mesh=_sc_mesh(),
        scratch_types=[
            pltpu.VMEM((kb, EB), jnp.int32),
            pltpu.VMEM((EB, d), jnp.float32),
            pltpu.VMEM_SHARED((npad, d), jnp.float32),
        ],
    )
    def deg_kernel(col_hbm, out_hbm, col_v, buf_v, acc_sh):
        c = lax.axis_index("c")
        s = lax.axis_index("s")
        wid = c * NS + s

        @pl.loop(0, EB)
        def _(i):
            for j in range(d // 16):
                buf_v[i, pl.ds(j * 16, 16)] = jnp.zeros((16,), jnp.float32)

        base = s * rpt
        nfull = rpt // EB
        rem = rpt - nfull * EB

        @pl.loop(0, nfull)
        def _(k):
            pltpu.sync_copy(buf_v, acc_sh.at[pl.ds(base + k * EB, EB)])

        if rem:
            pltpu.sync_copy(buf_v.at[pl.ds(0, rem)],
                            acc_sh.at[pl.ds(base + nfull * EB, rem)])

        @pl.loop(0, EB)
        def _(i):
            for j in range(d // 16):
                buf_v[i, pl.ds(j * 16, 16)] = jnp.ones((16,), jnp.float32)

        pltpu.sync_copy(col_hbm.at[wid], col_v)
        plsc.subcore_barrier()

        @pl.loop(0, kb)
        def _(j):
            pltpu.sync_copy(buf_v, acc_sh.at[col_v.at[j]], add=True)

        plsc.subcore_barrier()

        @pl.loop(0, nfull)
        def _(k):
            pltpu.sync_copy(acc_sh.at[pl.ds(base + k * EB, EB)],
                            out_hbm.at[c, pl.ds(base + k * EB, EB)])

        if rem:
            pltpu.sync_copy(acc_sh.at[pl.ds(base + nfull * EB, rem)],
                            out_hbm.at[c, pl.ds(base + nfull * EB, rem)])

    return deg_kernel


def _make_layer_kernel(npad, kb, d):
    rpt = npad // NS

    @functools.partial(
        pl.kernel,
        out_type=jax.ShapeDtypeStruct((NC, npad, d), jnp.float32),
        mesh=_sc_mesh(),
        scratch_types=[
            pltpu.VMEM((kb, EB), jnp.int32),
            pltpu.VMEM((kb, EB), jnp.int32),
            pltpu.VMEM((EB, d), jnp.float32),
            pltpu.VMEM_SHARED((npad, d), jnp.float32),
        ],
    )
    def layer_kernel(g_hbm, row_hbm, col_hbm, out_hbm, row_v, col_v, buf_v, acc_sh):
        c = lax.axis_index("c")
        s = lax.axis_index("s")
        wid = c * NS + s

        @pl.loop(0, EB)
        def _(i):
            for j in range(d // 16):
                buf_v[i, pl.ds(j * 16, 16)] = jnp.zeros((16,), jnp.float32)

        base = s * rpt
        nfull = rpt // EB
        rem = rpt - nfull * EB

        @pl.loop(0, nfull)
        def _(k):
            pltpu.sync_copy(buf_v, acc_sh.at[pl.ds(base + k * EB, EB)])

        if rem:
            pltpu.sync_copy(buf_v.at[pl.ds(0, rem)],
                            acc_sh.at[pl.ds(base + nfull * EB, rem)])

        pltpu.sync_copy(row_hbm.at[wid], row_v)
        pltpu.sync_copy(col_hbm.at[wid], col_v)
        plsc.subcore_barrier()

        @pl.loop(0, kb)
        def _(j):
            pltpu.sync_copy(g_hbm.at[row_v.at[j]], buf_v)
            pltpu.sync_copy(buf_v, acc_sh.at[col_v.at[j]], add=True)

        plsc.subcore_barrier()

        @pl.loop(0, nfull)
        def _(k):
            pltpu.sync_copy(acc_sh.at[pl.ds(base + k * EB, EB)],
                            out_hbm.at[c, pl.ds(base + k * EB, EB)])

        if rem:
            pltpu.sync_copy(acc_sh.at[pl.ds(base + nfull * EB, rem)],
                            out_hbm.at[c, pl.ds(base + nfull * EB, rem)])

    return layer_kernel


def _deg_combine_body(pa_ref, pb_ref, u_ref, dv_ref, g_ref):
    dinv = lax.rsqrt(pa_ref[...] + pb_ref[...] + 1.0)
    dv_ref[...] = dinv
    g_ref[...] = dinv * u_ref[...]


def _tc_mm_body(x_ref, w_ref, o_ref):
    o_ref[...] = jnp.dot(x_ref[...], w_ref[...],
                         preferred_element_type=jnp.float32)


def _tc_mid_body(sa_ref, sb_ref, g_ref, dv_ref, b_ref, w_ref, o_ref):
    dinv = dv_ref[...]
    h = dinv * (sa_ref[...] + sb_ref[...] + g_ref[...]) + b_ref[...]
    h = jnp.maximum(h, 0.0)
    o_ref[...] = dinv * jnp.dot(h, w_ref[...], preferred_element_type=jnp.float32)


def _tc_last_body(sa_ref, sb_ref, g_ref, dv_ref, b_ref, o_ref):
    o_ref[...] = dv_ref[...] * (sa_ref[...] + sb_ref[...] + g_ref[...]) + b_ref[...]


def _row_spec(r, d):
    return pl.BlockSpec((r, d), lambda i: (i, 0))


def _full_spec(shape):
    return pl.BlockSpec(shape, lambda i: tuple(0 for _ in shape))


def _tc_call(body, in_specs, grid, npad, d, args):
    return pl.pallas_call(
        body,
        grid=(grid,),
        in_specs=in_specs,
        out_specs=_row_spec(npad // grid, d),
        out_shape=jax.ShapeDtypeStruct((npad, d), jnp.float32),
    )(*args)


def kernel(x, edge_index, W1, b1, W2, b2, W3, b3):
    n, d_in = x.shape
    e = edge_index.shape[1]
    d = W1.shape[1]

    npad = ((n + 1 + 127) // 128) * 128
    e_pad = ((e + NW * EB - 1) // (NW * EB)) * (NW * EB)
    kb = e_pad // (NW * EB)

    row = edge_index[0]
    col = edge_index[1]
    pad = e_pad - e
    row_p = jnp.concatenate([row, jnp.zeros((pad,), jnp.int32)]).reshape(NW, kb, EB)
    col_p = jnp.concatenate([col, jnp.full((pad,), n, jnp.int32)]).reshape(NW, kb, EB)
    x_p = jnp.pad(x, ((0, npad - n), (0, 0)))

    deg_kernel = _make_deg_kernel(npad, kb, d)
    layer_kernel = _make_layer_kernel(npad, kb, d)

    u1 = pl.pallas_call(
        _tc_mm_body,
        grid=(8,),
        in_specs=[_row_spec(npad // 8, d_in), _full_spec((d_in, d))],
        out_specs=_row_spec(npad // 8, d),
        out_shape=jax.ShapeDtypeStruct((npad, d), jnp.float32),
    )(x_p, W1)

    degp = deg_kernel(col_p)

    dinv_b, g1 = pl.pallas_call(
        _deg_combine_body,
        grid=(8,),
        in_specs=[_row_spec(npad // 8, d), _row_spec(npad // 8, d),
                  _row_spec(npad // 8, d)],
        out_specs=[_row_spec(npad // 8, d), _row_spec(npad // 8, d)],
        out_shape=[jax.ShapeDtypeStruct((npad, d), jnp.float32),
                   jax.ShapeDtypeStruct((npad, d), jnp.float32)],
    )(degp[0], degp[1], u1)

    grid = 8
    r = npad // grid
    b1r = b1.reshape(1, d)
    b2r = b2.reshape(1, d)
    b3r = b3.reshape(1, d)

    dv_spec = _row_spec(r, 128)
    w_spec = _full_spec((d_in, d))
    b_spec = _full_spec((1, d))

    s1 = layer_kernel(g1, row_p, col_p)
    g2 = _tc_call(_tc_mid_body,
                  [_row_spec(r, d), _row_spec(r, d), _row_spec(r, d),
                   dv_spec, b_spec, w_spec],
                  grid, npad, d, (s1[0], s1[1], g1, dinv_b, b1r, W2))
    s2 = layer_kernel(g2, row_p, col_p)
    g3 = _tc_call(_tc_mid_body,
                  [_row_spec(r, d), _row_spec(r, d), _row_spec(r, d),
                   dv_spec, b_spec, w_spec],
                  grid, npad, d, (s2[0], s2[1], g2, dinv_b, b2r, W3))
    s3 = layer_kernel(g3, row_p, col_p)
    out = _tc_call(_tc_last_body,
                   [_row_spec(r, d), _row_spec(r, d), _row_spec(r, d),
                    dv_spec, b_spec],
                   grid, npad, d, (s3[0], s3[1], g3, dinv_b, b3r))
    return out[:n]
